# baseline (speedup 1.0000x reference)
import functools

import jax
import jax.numpy as jnp
from jax import lax
from jax.experimental import pallas as pl
from jax.experimental.pallas import tpu as pltpu
from jax.experimental.pallas import tpu_sc as plsc

_N = 10000
_E = 320000
_D = 128
_ALPHA = 0.9

_NC = 2
_NS = 16
_NW = _NC * _NS
_CH = 96
_Q = 27
_NQ = 4
_NCH = _Q * _NQ
_PER_W = _NCH * _CH
_E_PAD = _NW * _PER_W
_N_PAD = 10240
_RPW = _N_PAD // _NS
_ZR = 80


def _sc_body(x_hbm, src_hbm, dst_hbm, val_hbm, out_hbm,
             agg_sh, src_v, dst_v, val_v, rows0, rows1, rows2,
             gsem0, gsem1, gsem2, ssem0, ssem1, ssem2):
    c = lax.axis_index("c")
    s = lax.axis_index("s")
    w = c * _NS + s

    rows = (rows0, rows1, rows2)
    gsem = (gsem0, gsem1, gsem2)
    ssem = (ssem0, ssem1, ssem2)

    def zset(i, carry):
        z = jnp.zeros((16,), jnp.float32)
        for f in range(_D // 16):
            rows0[i, pl.ds(f * 16, 16)] = z
        return carry

    lax.fori_loop(0, _ZR, zset, 0)

    def zdma(k, carry):
        pltpu.sync_copy(rows0.at[pl.ds(0, _ZR)],
                        agg_sh.at[pl.ds(s * _RPW + k * _ZR, _ZR)])
        return carry

    lax.fori_loop(0, _RPW // _ZR, zdma, 0)

    plsc.subcore_barrier()

    def g_issue(j, b):
        pltpu.async_copy(x_hbm.at[src_v.at[j]], rows[b], gsem[b])

    def g_wait(j, b):
        pltpu.make_async_copy(x_hbm.at[src_v.at[j]], rows[b], gsem[b]).wait()

    def s_issue(j, b):
        pltpu.async_copy(rows[b], agg_sh.at[dst_v.at[j]], ssem[b], add=True)

    def s_wait(j, b):
        pltpu.make_async_copy(rows[b], agg_sh.at[dst_v.at[j]], ssem[b]).wait()

    def scale(j, b):
        rv = rows[b]

        def group(g, inner):
            vv16 = val_v[j, pl.ds(g * 16, 16)]
            for e16 in range(16):
                bc = jnp.take_along_axis(
                    vv16, jnp.full((16,), e16, jnp.int32), axis=0)
                e = g * 16 + e16
                for f in range(_D // 16):
                    sl = pl.ds(f * 16, 16)
                    rv[e, sl] = rv[e, sl] * bc
            return inner

        lax.fori_loop(0, _CH // 16, group, 0)

    def mid(j, b, bm1):
        g_wait(j, b)
        s_wait(j - 1, bm1)
        g_issue(j + 2, bm1)
        scale(j, b)
        s_issue(j, b)

    def block(qi, carry):
        pltpu.sync_copy(src_hbm.at[w, qi], src_v)
        pltpu.sync_copy(dst_hbm.at[w, qi], dst_v)
        pltpu.sync_copy(val_hbm.at[w, qi], val_v)

        g_issue(0, 0)
        g_issue(1, 1)
        g_wait(0, 0)
        g_issue(2, 2)
        scale(0, 0)
        s_issue(0, 0)
        g_wait(1, 1)
        s_wait(0, 0)
        g_issue(3, 0)
        scale(1, 1)
        s_issue(1, 1)

        def triple(m, inner):
            j0 = 3 * m + 2
            mid(j0, 2, 1)
            mid(j0 + 1, 0, 2)
            mid(j0 + 2, 1, 0)
            return inner

        lax.fori_loop(0, (_Q - 6) // 3, triple, 0)
        mid(_Q - 4, 2, 1)
        mid(_Q - 3, 0, 2)
        g_wait(_Q - 2, 1)
        s_wait(_Q - 3, 0)
        scale(_Q - 2, 1)
        s_issue(_Q - 2, 1)
        g_wait(_Q - 1, 2)
        s_wait(_Q - 2, 1)
        scale(_Q - 1, 2)
        pltpu.sync_copy(rows[2], agg_sh.at[dst_v.at[_Q - 1]], add=True)
        return carry

    lax.fori_loop(0, _NQ, block, 0)

    plsc.subcore_barrier()

    pltpu.sync_copy(agg_sh.at[pl.ds(s * _RPW, _RPW)],
                    out_hbm.at[c, pl.ds(s * _RPW, _RPW)])


_sc_gather_scatter = functools.partial(
    pl.kernel,
    out_type=jax.ShapeDtypeStruct((_NC, _N_PAD, _D), jnp.float32),
    mesh=plsc.VectorSubcoreMesh(core_axis_name="c", subcore_axis_name="s"),
    scratch_types=[
        pltpu.VMEM_SHARED((_N_PAD, _D), jnp.float32),
        pltpu.VMEM((_Q, _CH), jnp.int32),
        pltpu.VMEM((_Q, _CH), jnp.int32),
        pltpu.VMEM((_Q, _CH), jnp.float32),
        pltpu.VMEM((_CH, _D), jnp.float32),
        pltpu.VMEM((_CH, _D), jnp.float32),
        pltpu.VMEM((_CH, _D), jnp.float32),
        pltpu.SemaphoreType.DMA,
        pltpu.SemaphoreType.DMA,
        pltpu.SemaphoreType.DMA,
        pltpu.SemaphoreType.DMA,
        pltpu.SemaphoreType.DMA,
        pltpu.SemaphoreType.DMA,
    ],
)(_sc_body)


_BLK = 400


def _tc_body(p_ref, xi_ref, wi_ref, wx_ref, o_ref):
    agg = p_ref[0] + p_ref[1]
    h = _ALPHA * agg + (1.0 - _ALPHA) * jnp.dot(
        xi_ref[...], wi_ref[...], preferred_element_type=jnp.float32)
    o_ref[...] = jnp.maximum(
        jnp.dot(h, wx_ref[...], preferred_element_type=jnp.float32), 0.0)


def _tc_dense(partials, xi, wi, wx):
    nblk = _N // _BLK
    return pl.pallas_call(
        _tc_body,
        out_shape=jax.ShapeDtypeStruct((_N, _D), jnp.float32),
        grid=(nblk,),
        in_specs=[
            pl.BlockSpec((2, _BLK, _D), lambda i: (0, i, 0)),
            pl.BlockSpec((_BLK, 8), lambda i: (i, 0)),
            pl.BlockSpec((8, _D), lambda i: (0, 0)),
            pl.BlockSpec((_D, _D), lambda i: (0, 0)),
        ],
        out_specs=pl.BlockSpec((_BLK, _D), lambda i: (i, 0)),
    )(partials, xi, wi, wx)


def kernel(x, x_initial, edge_index, adj_values, w_init, w_x):
    dst = edge_index[0]
    src = edge_index[1]
    pad = _E_PAD - _E
    zi = jnp.zeros((pad,), jnp.int32)
    srcp = jnp.concatenate([src, zi]).reshape(_NW, _NQ, _Q, _CH)
    dstp = jnp.concatenate([dst, zi]).reshape(_NW, _NQ, _Q, _CH)
    valp = jnp.concatenate(
        [adj_values, jnp.zeros((pad,), jnp.float32)]).reshape(_NW, _NQ, _Q, _CH)

    partials = _sc_gather_scatter(x, srcp, dstp, valp)

    xi = jnp.pad(x_initial, ((0, 0), (0, 5)))
    wi = jnp.pad(w_init, ((0, 5), (0, 0)))
    return _tc_dense(partials, xi, wi, w_x)


# speedup vs baseline: 1.1316x; 1.1316x over previous
import functools

import jax
import jax.numpy as jnp
import numpy as np
from jax import lax
from jax.experimental import pallas as pl
from jax.experimental.pallas import tpu as pltpu
from jax.experimental.pallas import tpu_sc as plsc

_N = 10000
_E = 320000
_D = 128
_DW = _D // 2
_ALPHA = 0.9

_NC = 2
_NS = 16
_NW = _NC * _NS
_CH = 128
_Q = 20
_NQ = 4
_NCH = _Q * _NQ
_PER_W = _NCH * _CH
_E_PAD = _NW * _PER_W
_N_PAD = 10240
_RPW = _N_PAD // _NS
_ZR = 128

_PERM = np.concatenate(
    [u * 32 + np.concatenate([np.arange(0, 32, 2), np.arange(1, 32, 2)])
     for u in range(_D // 32)])


def _sc_body(x_hbm, src_hbm, dst_hbm, val_hbm, out_hbm,
             agg_sh, src_v, dst_v, val_v, rows_pk, scaled_v):
    c = lax.axis_index("c")
    s = lax.axis_index("s")
    w = c * _NS + s

    def zset(i, carry):
        z = jnp.zeros((16,), jnp.float32)
        for f in range(_D // 16):
            scaled_v[i, pl.ds(f * 16, 16)] = z
        return carry

    lax.fori_loop(0, _ZR, zset, 0)

    def zdma(k, carry):
        pltpu.sync_copy(scaled_v, agg_sh.at[pl.ds(s * _RPW + k * _ZR, _ZR)])
        return carry

    lax.fori_loop(0, _RPW // _ZR, zdma, 0)

    plsc.subcore_barrier()

    hi_mask = jnp.full((16,), -65536, jnp.int32)

    def block(qi, carry):
        pltpu.sync_copy(src_hbm.at[w, qi], src_v)
        pltpu.sync_copy(dst_hbm.at[w, qi], dst_v)
        pltpu.sync_copy(val_hbm.at[w, qi], val_v)

        def chunk(i, inner):
            pltpu.sync_copy(x_hbm.at[src_v.at[i]], rows_pk)

            def group(g, acc):
                vv16 = val_v[i, pl.ds(g * 16, 16)]
                for e16 in range(16):
                    bc = jnp.take_along_axis(
                        vv16, jnp.full((16,), e16, jnp.int32), axis=0)
                    e = g * 16 + e16
                    for u in range(_D // 32):
                        wv = rows_pk[e, pl.ds(u * 16, 16)]
                        lo = jax.lax.bitcast_convert_type(wv << 16, jnp.float32)
                        hi = jax.lax.bitcast_convert_type(wv & hi_mask, jnp.float32)
                        scaled_v[e, pl.ds(u * 32, 16)] = lo * bc
                        scaled_v[e, pl.ds(u * 32 + 16, 16)] = hi * bc
                return acc

            lax.fori_loop(0, _CH // 16, group, 0)
            pltpu.sync_copy(scaled_v, agg_sh.at[dst_v.at[i]], add=True)
            return inner

        lax.fori_loop(0, _Q, chunk, 0)
        return carry

    lax.fori_loop(0, _NQ, block, 0)

    plsc.subcore_barrier()

    pltpu.sync_copy(agg_sh.at[pl.ds(s * _RPW, _RPW)],
                    out_hbm.at[c, pl.ds(s * _RPW, _RPW)])


_sc_gather_scatter = functools.partial(
    pl.kernel,
    out_type=jax.ShapeDtypeStruct((_NC, _N_PAD, _D), jnp.float32),
    mesh=plsc.VectorSubcoreMesh(core_axis_name="c", subcore_axis_name="s"),
    compiler_params=pltpu.CompilerParams(use_tc_tiling_on_sc=False),
    scratch_types=[
        pltpu.VMEM_SHARED((_N_PAD, _D), jnp.float32),
        pltpu.VMEM((_Q, _CH), jnp.int32),
        pltpu.VMEM((_Q, _CH), jnp.int32),
        pltpu.VMEM((_Q, _CH), jnp.float32),
        pltpu.VMEM((_CH, _DW), jnp.int32),
        pltpu.VMEM((_CH, _D), jnp.float32),
    ],
)(_sc_body)


_BLK = 400


def _tc_body(p_ref, xi_ref, wi_ref, wx_ref, o_ref):
    agg = p_ref[0] + p_ref[1]
    h = _ALPHA * agg + (1.0 - _ALPHA) * jnp.dot(
        xi_ref[...], wi_ref[...], preferred_element_type=jnp.float32)
    o_ref[...] = jnp.maximum(
        jnp.dot(h, wx_ref[...], preferred_element_type=jnp.float32), 0.0)


def _tc_dense(partials, xi, wi, wx):
    nblk = _N // _BLK
    return pl.pallas_call(
        _tc_body,
        out_shape=jax.ShapeDtypeStruct((_N, _D), jnp.float32),
        grid=(nblk,),
        in_specs=[
            pl.BlockSpec((_NC, _BLK, _D), lambda i: (0, i, 0)),
            pl.BlockSpec((_BLK, 8), lambda i: (i, 0)),
            pl.BlockSpec((8, _D), lambda i: (0, 0)),
            pl.BlockSpec((_D, _D), lambda i: (0, 0)),
        ],
        out_specs=pl.BlockSpec((_BLK, _D), lambda i: (i, 0)),
    )(partials, xi, wi, wx)


def kernel(x, x_initial, edge_index, adj_values, w_init, w_x):
    dst = edge_index[0]
    src = edge_index[1]
    pad = _E_PAD - _E
    zi = jnp.zeros((pad,), jnp.int32)
    srcp = jnp.concatenate([src, zi]).reshape(_NW, _NQ, _Q, _CH)
    dstp = jnp.concatenate([dst, zi]).reshape(_NW, _NQ, _Q, _CH)
    valp = jnp.concatenate(
        [adj_values, jnp.zeros((pad,), jnp.float32)]).reshape(_NW, _NQ, _Q, _CH)

    x_pk = jax.lax.bitcast_convert_type(
        x.astype(jnp.bfloat16).reshape(_N, _DW, 2), jnp.int32)

    partials = _sc_gather_scatter(x_pk, srcp, dstp, valp)

    xi = jnp.pad(x_initial, ((0, 0), (0, 5)))
    wi = jnp.pad(w_init, ((0, 5), (0, 0)))[:, _PERM]
    wxp = w_x[_PERM, :]
    return _tc_dense(partials, xi, wi, wxp)


# speedup vs baseline: 1.1558x; 1.0214x over previous
import functools

import jax
import jax.numpy as jnp
import numpy as np
from jax import lax
from jax.experimental import pallas as pl
from jax.experimental.pallas import tpu as pltpu
from jax.experimental.pallas import tpu_sc as plsc

_N = 10000
_E = 320000
_D = 128
_DW = _D // 2
_ALPHA = 0.9

_NC = 2
_NS = 16
_NW = _NC * _NS
_CH = 128
_Q = 20
_NQ = 4
_NCH = _Q * _NQ
_PER_W = _NCH * _CH
_E_PAD = _NW * _PER_W
_N_PAD = 10240
_RPW = _N_PAD // _NS
_ZR = 128

_PERM = np.concatenate(
    [u * 32 + np.concatenate([np.arange(0, 32, 2), np.arange(1, 32, 2)])
     for u in range(_D // 32)])


def _sc_body(x_hbm, src_hbm, dst_hbm, val_hbm, out_hbm,
             agg_sh, src_v, dst_v, val_v, scaled_v):
    c = lax.axis_index("c")
    s = lax.axis_index("s")
    w = c * _NS + s

    def zset(i, carry):
        z = jnp.zeros((16,), jnp.float32)
        for f in range(_D // 16):
            scaled_v[i, pl.ds(f * 16, 16)] = z
        return carry

    lax.fori_loop(0, _ZR, zset, 0)

    def zdma(k, carry):
        pltpu.sync_copy(scaled_v, agg_sh.at[pl.ds(s * _RPW + k * _ZR, _ZR)])
        return carry

    lax.fori_loop(0, _RPW // _ZR, zdma, 0)

    plsc.subcore_barrier()

    hi_mask = jnp.full((16,), -65536, jnp.int32)

    def block(qi, carry):
        pltpu.sync_copy(src_hbm.at[w, qi], src_v)
        pltpu.sync_copy(dst_hbm.at[w, qi], dst_v)
        pltpu.sync_copy(val_hbm.at[w, qi], val_v)

        def chunk(i, inner):
            pltpu.sync_copy(x_hbm.at[src_v.at[i]], scaled_v)

            def group(g, acc):
                vv16 = val_v[i, pl.ds(g * 16, 16)]
                for e16 in range(16):
                    bc = jnp.take_along_axis(
                        vv16, jnp.full((16,), e16, jnp.int32), axis=0)
                    e = g * 16 + e16
                    for f in range(_D // 16):
                        sl = pl.ds(f * 16, 16)
                        scaled_v[e, sl] = scaled_v[e, sl] * bc
                return acc

            lax.fori_loop(0, _CH // 16, group, 0)
            pltpu.sync_copy(scaled_v, agg_sh.at[dst_v.at[i]], add=True)
            return inner

        lax.fori_loop(0, _Q, chunk, 0)
        return carry

    lax.fori_loop(0, _NQ, block, 0)

    plsc.subcore_barrier()

    pltpu.sync_copy(agg_sh.at[pl.ds(s * _RPW, _RPW)],
                    out_hbm.at[c, pl.ds(s * _RPW, _RPW)])


_sc_gather_scatter = functools.partial(
    pl.kernel,
    out_type=jax.ShapeDtypeStruct((_NC, _N_PAD, _D), jnp.float32),
    mesh=plsc.VectorSubcoreMesh(core_axis_name="c", subcore_axis_name="s"),
    compiler_params=pltpu.CompilerParams(use_tc_tiling_on_sc=False),
    scratch_types=[
        pltpu.VMEM_SHARED((_N_PAD, _D), jnp.float32),
        pltpu.VMEM((_Q, _CH), jnp.int32),
        pltpu.VMEM((_Q, _CH), jnp.int32),
        pltpu.VMEM((_Q, _CH), jnp.float32),
        pltpu.VMEM((_CH, _D), jnp.float32),
    ],
)(_sc_body)


_BLK = 400


def _tc_body(p_ref, xi_ref, wi_ref, wx_ref, o_ref):
    agg = p_ref[0] + p_ref[1]
    h = _ALPHA * agg + (1.0 - _ALPHA) * jnp.dot(
        xi_ref[...], wi_ref[...], preferred_element_type=jnp.float32)
    o_ref[...] = jnp.maximum(
        jnp.dot(h, wx_ref[...], preferred_element_type=jnp.float32), 0.0)


def _tc_dense(partials, xi, wi, wx):
    nblk = _N // _BLK
    return pl.pallas_call(
        _tc_body,
        out_shape=jax.ShapeDtypeStruct((_N, _D), jnp.float32),
        grid=(nblk,),
        in_specs=[
            pl.BlockSpec((_NC, _BLK, _D), lambda i: (0, i, 0)),
            pl.BlockSpec((_BLK, 8), lambda i: (i, 0)),
            pl.BlockSpec((8, _D), lambda i: (0, 0)),
            pl.BlockSpec((_D, _D), lambda i: (0, 0)),
        ],
        out_specs=pl.BlockSpec((_BLK, _D), lambda i: (i, 0)),
    )(partials, xi, wi, wx)


def kernel(x, x_initial, edge_index, adj_values, w_init, w_x):
    dst = edge_index[0]
    src = edge_index[1]
    pad = _E_PAD - _E
    zi = jnp.zeros((pad,), jnp.int32)
    srcp = jnp.concatenate([src, zi]).reshape(_NW, _NQ, _Q, _CH)
    dstp = jnp.concatenate([dst, zi]).reshape(_NW, _NQ, _Q, _CH)
    valp = jnp.concatenate(
        [adj_values, jnp.zeros((pad,), jnp.float32)]).reshape(_NW, _NQ, _Q, _CH)

    partials = _sc_gather_scatter(x, srcp, dstp, valp)

    xi = jnp.pad(x_initial, ((0, 0), (0, 5)))
    wi = jnp.pad(w_init, ((0, 5), (0, 0)))
    return _tc_dense(partials, xi, wi, w_x)
